# baseline (speedup 1.0000x reference)
import functools

import jax
import jax.numpy as jnp
from jax import lax
from jax.experimental import pallas as pl
from jax.experimental.pallas import tpu as pltpu
from jax.experimental.pallas import tpu_sc as plsc

B = 16384
D = 32
L = 16
NUM_CORES = 2
NUM_SUBCORES = 16
NW = NUM_CORES * NUM_SUBCORES
BPW = B // NW
CH = 4
NCH = BPW // CH
HCH = NCH // 2
HALF = D // 2
NMT = 782
MPK = NMT * 32
MB = 4
NB = 13

_mesh = plsc.VectorSubcoreMesh(core_axis_name="c", subcore_axis_name="s")


@functools.partial(
    pl.kernel,
    mesh=_mesh,
    out_type=(
        jax.ShapeDtypeStruct((B,), jnp.float32),
        jax.ShapeDtypeStruct((NUM_CORES, MPK, 128), jnp.float32),
    ),
    scratch_types=[
        pltpu.VMEM((BPW + L,), jnp.int32),
        pltpu.VMEM((BPW + L,), jnp.int32),
        pltpu.VMEM((BPW,), jnp.int32),
        pltpu.VMEM((2 * CH * D, 128), jnp.float32),
        pltpu.VMEM((MB * D, 128), jnp.float32),
        pltpu.VMEM((MB * D, 128), jnp.float32),
        pltpu.VMEM((BPW // 2, 128), jnp.float32),
        pltpu.VMEM((HALF * BPW,), jnp.float32),
        pltpu.VMEM((BPW,), jnp.float32),
        pltpu.VMEM((BPW,), jnp.float32),
        pltpu.VMEM((BPW,), jnp.float32),
        pltpu.SemaphoreType.DMA,
        pltpu.SemaphoreType.DMA,
        pltpu.SemaphoreType.DMA,
        pltpu.SemaphoreType.DMA,
    ],
    compiler_params=pltpu.CompilerParams(
        needs_layout_passes=False,
        disable_bounds_checks=True,
    ),
)
def _mf_sc(user_hbm, mission_hbm, uembT_hbm, membT_hbm, ubias_hbm, mbias_hbm,
           out_hbm, mscr_hbm, uidx_v, midx_v, mpk_v, ublk_v, inblk_v,
           outblk_v, mrows_v, prod_v, ub_v, mb_v, o_v,
           sem, sem_a, sem_b, sem_out):
    cid = lax.axis_index("c")
    sid = lax.axis_index("s")
    wid = sid * NUM_CORES + cid
    base = wid * BPW

    pltpu.sync_copy(user_hbm.at[pl.ds(base, BPW)], uidx_v.at[pl.ds(0, BPW)])
    pltpu.sync_copy(mission_hbm.at[pl.ds(base, BPW)], midx_v.at[pl.ds(0, BPW)])
    uidx_v[pl.ds(BPW, L)] = jnp.zeros((L,), jnp.int32)
    midx_v[pl.ds(BPW, L)] = jnp.zeros((L,), jnp.int32)

    cp_ub = pltpu.async_copy(ubias_hbm.at[uidx_v.at[pl.ds(0, BPW)]], ub_v, sem)
    cp_mb = pltpu.async_copy(mbias_hbm.at[midx_v.at[pl.ds(0, BPW)]], mb_v, sem)

    iota = lax.iota(jnp.int32, L)

    t0 = jnp.minimum(sid * 49, NMT - MB * NB)

    def relayout_body(b, carry):
        tb = t0 + b * MB
        for i in range(MB):
            off = pl.multiple_of((tb + i) * 128, 128)
            pltpu.async_copy(membT_hbm.at[:, pl.ds(off, 128)],
                             inblk_v.at[pl.ds(i * D, D)], sem_a)
        for i in range(MB):
            pltpu.make_async_copy(membT_hbm.at[:, pl.ds(0, 128)],
                                  inblk_v.at[pl.ds(i * D, D)], sem_a).wait()

        @pl.when(b >= 1)
        def _():
            pltpu.make_async_copy(outblk_v,
                                  mscr_hbm.at[cid, pl.ds(0, MB * D), :],
                                  sem_out).wait()

        def q_body(q2, carry2):
            ti = q2 >> 5
            q = q2 & 31
            for cg in range(8):
                d_vec = ti * D + iota + (cg & 1) * L
                src_lane = jnp.broadcast_to(4 * q + cg // 2, (L,))
                val = plsc.load_gather(inblk_v, [d_vec, src_lane])
                plsc.store_scatter(
                    outblk_v,
                    [jnp.broadcast_to(q2, (L,)), cg * L + iota],
                    val)
            return carry2

        lax.fori_loop(0, MB * 32, q_body, 0)
        pltpu.async_copy(outblk_v, mscr_hbm.at[cid, pl.ds(tb * 32, MB * D), :],
                         sem_out)
        return carry

    lax.fori_loop(0, NB, relayout_body, 0)
    pltpu.make_async_copy(outblk_v, mscr_hbm.at[cid, pl.ds(0, MB * D), :],
                          sem_out).wait()

    def pack_body(i, carry):
        sl = pl.ds(i * L, L)
        mpk_v[sl] = lax.shift_right_logical(midx_v[sl], 2)
        return carry

    lax.fori_loop(0, BPW // L, pack_body, 0)
    cp_ub.wait()
    cp_mb.wait()
    plsc.subcore_barrier()

    def _fire(c, slot, semx):
        uvec = uidx_v[pl.ds(c * CH, L)]
        for k in range(CH):
            ut = pl.multiple_of((uvec[k] >> 7) * 128, 128)
            row = (slot * CH + k) * D
            pltpu.async_copy(uembT_hbm.at[:, pl.ds(ut, 128)],
                             ublk_v.at[pl.ds(row, D)], semx)

    def _drain(slot, semx):
        for k in range(CH):
            row = (slot * CH + k) * D
            pltpu.make_async_copy(uembT_hbm.at[:, pl.ds(0, 128)],
                                  ublk_v.at[pl.ds(row, D)], semx).wait()

    def chunk_body(c, carry):
        even = (c & 1) == 0
        more_u = c + 1 < NCH

        @pl.when(jnp.logical_and(even, more_u))
        def _():
            _fire(c + 1, 1, sem_b)

        @pl.when(jnp.logical_and(jnp.logical_not(even), more_u))
        def _():
            _fire(c + 1, 0, sem_a)

        @pl.when(even)
        def _():
            _drain(0, sem_a)

        @pl.when(jnp.logical_not(even))
        def _():
            _drain(1, sem_b)

        srow = (c & 1) * CH * D
        uvec = uidx_v[pl.ds(c * CH, L)]
        mvec = midx_v[pl.ds(c * CH, L)]
        for k in range(CH):
            e = c * CH + k
            ulane = jnp.broadcast_to(uvec[k] & 127, (L,))
            erow = jnp.broadcast_to(e & (BPW // 2 - 1), (L,))
            mcol = (mvec[k] & 3) * D + iota
            u_lo = plsc.load_gather(ublk_v, [srow + k * D + iota, ulane])
            u_hi = plsc.load_gather(ublk_v, [srow + k * D + HALF + iota, ulane])
            m_lo = plsc.load_gather(mrows_v, [erow, mcol])
            m_hi = plsc.load_gather(mrows_v, [erow, mcol + HALF])
            p = u_lo * m_lo + u_hi * m_hi
            plsc.store_scatter(prod_v, [iota * BPW + e], p)
        return carry

    _fire(0, 0, sem_a)
    pltpu.async_copy(mscr_hbm.at[cid].at[mpk_v.at[pl.ds(0, BPW // 2)]],
                     mrows_v, sem).wait()
    lax.fori_loop(0, HCH, chunk_body, 0)
    pltpu.async_copy(mscr_hbm.at[cid].at[mpk_v.at[pl.ds(BPW // 2, BPW // 2)]],
                     mrows_v, sem).wait()
    lax.fori_loop(HCH, NCH, chunk_body, 0)

    def group_body(g, carry):
        sl = pl.ds(g * L, L)
        acc = ub_v[sl] + mb_v[sl]
        for j in range(HALF):
            acc = acc + prod_v[pl.ds(j * BPW + g * L, L)]
        o_v[sl] = acc
        return carry

    lax.fori_loop(0, BPW // L, group_body, 0)

    pltpu.sync_copy(o_v, out_hbm.at[pl.ds(base, BPW)])


def kernel(user, mission, user_embedding, mission_embedding, user_bias, mission_bias):
    uembT = user_embedding.T
    membT = mission_embedding.T
    ub = user_bias.reshape(-1)
    mb = mission_bias.reshape(-1)
    out, _ = _mf_sc(user, mission, uembT, membT, ub, mb)
    return out


# speedup vs baseline: 1.0238x; 1.0238x over previous
import functools

import jax
import jax.numpy as jnp
from jax import lax
from jax.experimental import pallas as pl
from jax.experimental.pallas import tpu as pltpu
from jax.experimental.pallas import tpu_sc as plsc

B = 16384
D = 32
L = 16
NUM_CORES = 2
NUM_SUBCORES = 16
NW = NUM_CORES * NUM_SUBCORES
BPW = B // NW
CH = 4
NCH = BPW // CH
HCH = NCH // 2
HALF = D // 2
NMT = 782
MPK = NMT * 32
MB = 4
NB = 13

_mesh = plsc.VectorSubcoreMesh(core_axis_name="c", subcore_axis_name="s")


@functools.partial(
    pl.kernel,
    mesh=_mesh,
    out_type=(
        jax.ShapeDtypeStruct((B,), jnp.float32),
        jax.ShapeDtypeStruct((NUM_CORES, MPK, 128), jnp.float32),
    ),
    scratch_types=[
        pltpu.VMEM((BPW + L,), jnp.int32),
        pltpu.VMEM((BPW + L,), jnp.int32),
        pltpu.VMEM((BPW,), jnp.int32),
        pltpu.VMEM((2 * CH * D, 128), jnp.float32),
        pltpu.VMEM((2 * MB * D, 128), jnp.float32),
        pltpu.VMEM((MB * D, 128), jnp.float32),
        pltpu.VMEM((BPW // 2, 128), jnp.float32),
        pltpu.VMEM((HALF * BPW,), jnp.float32),
        pltpu.VMEM((BPW,), jnp.float32),
        pltpu.VMEM((BPW,), jnp.float32),
        pltpu.VMEM((BPW,), jnp.float32),
        pltpu.SemaphoreType.DMA,
        pltpu.SemaphoreType.DMA,
        pltpu.SemaphoreType.DMA,
        pltpu.SemaphoreType.DMA,
    ],
    compiler_params=pltpu.CompilerParams(
        needs_layout_passes=False,
        disable_bounds_checks=True,
    ),
)
def _mf_sc(user_hbm, mission_hbm, uembT_hbm, membT_hbm, ubias_hbm, mbias_hbm,
           out_hbm, mscr_hbm, uidx_v, midx_v, mpk_v, ublk_v, inblk_v,
           outblk_v, mrows_v, prod_v, ub_v, mb_v, o_v,
           sem, sem_a, sem_b, sem_out):
    cid = lax.axis_index("c")
    sid = lax.axis_index("s")
    wid = sid * NUM_CORES + cid
    base = wid * BPW

    pltpu.sync_copy(user_hbm.at[pl.ds(base, BPW)], uidx_v.at[pl.ds(0, BPW)])
    pltpu.sync_copy(mission_hbm.at[pl.ds(base, BPW)], midx_v.at[pl.ds(0, BPW)])
    uidx_v[pl.ds(BPW, L)] = jnp.zeros((L,), jnp.int32)
    midx_v[pl.ds(BPW, L)] = jnp.zeros((L,), jnp.int32)

    cp_ub = pltpu.async_copy(ubias_hbm.at[uidx_v.at[pl.ds(0, BPW)]], ub_v, sem)
    cp_mb = pltpu.async_copy(mbias_hbm.at[midx_v.at[pl.ds(0, BPW)]], mb_v, sem)

    iota = lax.iota(jnp.int32, L)

    t0 = jnp.minimum(sid * 49, NMT - MB * NB)

    def _rfire(b, slot, semx):
        tb = t0 + b * MB
        for i in range(MB):
            off = pl.multiple_of((tb + i) * 128, 128)
            pltpu.async_copy(membT_hbm.at[:, pl.ds(off, 128)],
                             inblk_v.at[pl.ds((slot * MB + i) * D, D)], semx)

    def _rdrain(slot, semx):
        for i in range(MB):
            pltpu.make_async_copy(
                membT_hbm.at[:, pl.ds(0, 128)],
                inblk_v.at[pl.ds((slot * MB + i) * D, D)], semx).wait()

    _rfire(0, 0, sem_a)

    def relayout_body(b, carry):
        tb = t0 + b * MB
        even = (b & 1) == 0
        more = b + 1 < NB

        @pl.when(jnp.logical_and(even, more))
        def _():
            _rfire(b + 1, 1, sem_b)

        @pl.when(jnp.logical_and(jnp.logical_not(even), more))
        def _():
            _rfire(b + 1, 0, sem_a)

        @pl.when(even)
        def _():
            _rdrain(0, sem_a)

        @pl.when(jnp.logical_not(even))
        def _():
            _rdrain(1, sem_b)

        @pl.when(b >= 1)
        def _():
            pltpu.make_async_copy(outblk_v,
                                  mscr_hbm.at[cid, pl.ds(0, MB * D), :],
                                  sem_out).wait()

        srow = (b & 1) * MB * D
        for ti in range(MB):
            for q in range(32):
                for cg in range(8):
                    d_vec = srow + ti * D + iota + (cg & 1) * L
                    src_lane = jnp.broadcast_to(4 * q + cg // 2, (L,))
                    val = plsc.load_gather(inblk_v, [d_vec, src_lane])
                    outblk_v[ti * 32 + q, pl.ds(cg * L, L)] = val

        pltpu.async_copy(outblk_v, mscr_hbm.at[cid, pl.ds(tb * 32, MB * D), :],
                         sem_out)
        return carry

    lax.fori_loop(0, NB, relayout_body, 0)
    pltpu.make_async_copy(outblk_v, mscr_hbm.at[cid, pl.ds(0, MB * D), :],
                          sem_out).wait()

    def pack_body(i, carry):
        sl = pl.ds(i * L, L)
        mpk_v[sl] = lax.shift_right_logical(midx_v[sl], 2)
        return carry

    lax.fori_loop(0, BPW // L, pack_body, 0)
    cp_ub.wait()
    cp_mb.wait()
    plsc.subcore_barrier()

    def _fire(c, slot, semx):
        uvec = uidx_v[pl.ds(c * CH, L)]
        for k in range(CH):
            ut = pl.multiple_of((uvec[k] >> 7) * 128, 128)
            row = (slot * CH + k) * D
            pltpu.async_copy(uembT_hbm.at[:, pl.ds(ut, 128)],
                             ublk_v.at[pl.ds(row, D)], semx)

    def _drain(slot, semx):
        for k in range(CH):
            row = (slot * CH + k) * D
            pltpu.make_async_copy(uembT_hbm.at[:, pl.ds(0, 128)],
                                  ublk_v.at[pl.ds(row, D)], semx).wait()

    def chunk_body(c, carry):
        even = (c & 1) == 0
        more_u = c + 1 < NCH

        @pl.when(jnp.logical_and(even, more_u))
        def _():
            _fire(c + 1, 1, sem_b)

        @pl.when(jnp.logical_and(jnp.logical_not(even), more_u))
        def _():
            _fire(c + 1, 0, sem_a)

        @pl.when(even)
        def _():
            _drain(0, sem_a)

        @pl.when(jnp.logical_not(even))
        def _():
            _drain(1, sem_b)

        srow = (c & 1) * CH * D
        uvec = uidx_v[pl.ds(c * CH, L)]
        mvec = midx_v[pl.ds(c * CH, L)]
        for k in range(CH):
            e = c * CH + k
            ulane = jnp.broadcast_to(uvec[k] & 127, (L,))
            erow = jnp.broadcast_to(e & (BPW // 2 - 1), (L,))
            mcol = (mvec[k] & 3) * D + iota
            u_lo = plsc.load_gather(ublk_v, [srow + k * D + iota, ulane])
            u_hi = plsc.load_gather(ublk_v, [srow + k * D + HALF + iota, ulane])
            m_lo = plsc.load_gather(mrows_v, [erow, mcol])
            m_hi = plsc.load_gather(mrows_v, [erow, mcol + HALF])
            p = u_lo * m_lo + u_hi * m_hi
            plsc.store_scatter(prod_v, [iota * BPW + e], p)
        return carry

    _fire(0, 0, sem_a)
    pltpu.async_copy(mscr_hbm.at[cid].at[mpk_v.at[pl.ds(0, BPW // 2)]],
                     mrows_v, sem).wait()
    lax.fori_loop(0, HCH, chunk_body, 0)
    pltpu.async_copy(mscr_hbm.at[cid].at[mpk_v.at[pl.ds(BPW // 2, BPW // 2)]],
                     mrows_v, sem).wait()
    lax.fori_loop(HCH, NCH, chunk_body, 0)

    def group_body(g, carry):
        sl = pl.ds(g * L, L)
        acc = ub_v[sl] + mb_v[sl]
        for j in range(HALF):
            acc = acc + prod_v[pl.ds(j * BPW + g * L, L)]
        o_v[sl] = acc
        return carry

    lax.fori_loop(0, BPW // L, group_body, 0)

    pltpu.sync_copy(o_v, out_hbm.at[pl.ds(base, BPW)])


def kernel(user, mission, user_embedding, mission_embedding, user_bias, mission_bias):
    uembT = user_embedding.T
    membT = mission_embedding.T
    ub = user_bias.reshape(-1)
    mb = mission_bias.reshape(-1)
    out, _ = _mf_sc(user, mission, uembT, membT, ub, mb)
    return out


# speedup vs baseline: 1.2804x; 1.2507x over previous
import functools

import jax
import jax.numpy as jnp
from jax import lax
from jax.experimental import pallas as pl
from jax.experimental.pallas import tpu as pltpu
from jax.experimental.pallas import tpu_sc as plsc

B = 16384
D = 32
L = 16
NUM_CORES = 2
NUM_SUBCORES = 16
NW = NUM_CORES * NUM_SUBCORES
BPW = B // NW
CH = 4
NCH = BPW // CH
HALF = D // 2

_mesh = plsc.VectorSubcoreMesh(core_axis_name="c", subcore_axis_name="s")


@functools.partial(
    pl.kernel,
    mesh=_mesh,
    out_type=jax.ShapeDtypeStruct((B,), jnp.float32),
    scratch_types=[
        pltpu.VMEM((BPW + L,), jnp.int32),
        pltpu.VMEM((BPW + L,), jnp.int32),
        pltpu.VMEM((2 * CH * D, 128), jnp.float32),
        pltpu.VMEM((2 * CH * D, 128), jnp.float32),
        pltpu.VMEM((HALF * BPW,), jnp.float32),
        pltpu.VMEM((BPW,), jnp.float32),
        pltpu.VMEM((BPW,), jnp.float32),
        pltpu.VMEM((BPW,), jnp.float32),
        pltpu.SemaphoreType.DMA,
        pltpu.SemaphoreType.DMA,
        pltpu.SemaphoreType.DMA,
    ],
    compiler_params=pltpu.CompilerParams(
        needs_layout_passes=False,
        disable_bounds_checks=True,
    ),
)
def _mf_sc(user_hbm, mission_hbm, uembT_hbm, membT_hbm, ubias_hbm, mbias_hbm,
           out_hbm, uidx_v, midx_v, ublk_v, mblk_v, prod_v, ub_v, mb_v, o_v,
           sem, sem_a, sem_b):
    wid = lax.axis_index("s") * NUM_CORES + lax.axis_index("c")
    base = wid * BPW

    pltpu.sync_copy(user_hbm.at[pl.ds(base, BPW)], uidx_v.at[pl.ds(0, BPW)])
    pltpu.sync_copy(mission_hbm.at[pl.ds(base, BPW)], midx_v.at[pl.ds(0, BPW)])
    uidx_v[pl.ds(BPW, L)] = jnp.zeros((L,), jnp.int32)
    midx_v[pl.ds(BPW, L)] = jnp.zeros((L,), jnp.int32)

    cp_ub = pltpu.async_copy(ubias_hbm.at[uidx_v.at[pl.ds(0, BPW)]], ub_v, sem)
    cp_mb = pltpu.async_copy(mbias_hbm.at[midx_v.at[pl.ds(0, BPW)]], mb_v, sem)
    cp_ub.wait()
    cp_mb.wait()

    iota = lax.iota(jnp.int32, L)

    def _fire(c, slot, semx):
        uvec = uidx_v[pl.ds(c * CH, L)]
        mvec = midx_v[pl.ds(c * CH, L)]
        for k in range(CH):
            ut = pl.multiple_of((uvec[k] >> 7) * 128, 128)
            mt = pl.multiple_of((mvec[k] >> 7) * 128, 128)
            row = (slot * CH + k) * D
            pltpu.async_copy(uembT_hbm.at[:, pl.ds(ut, 128)],
                             ublk_v.at[pl.ds(row, D)], semx)
            pltpu.async_copy(membT_hbm.at[:, pl.ds(mt, 128)],
                             mblk_v.at[pl.ds(row, D)], semx)

    def _drain(slot, semx):
        for k in range(CH):
            row = (slot * CH + k) * D
            pltpu.make_async_copy(uembT_hbm.at[:, pl.ds(0, 128)],
                                  ublk_v.at[pl.ds(row, D)], semx).wait()
            pltpu.make_async_copy(membT_hbm.at[:, pl.ds(0, 128)],
                                  mblk_v.at[pl.ds(row, D)], semx).wait()

    _fire(0, 0, sem_a)

    def chunk_body(c, carry):
        even = (c & 1) == 0
        more = c + 1 < NCH

        @pl.when(jnp.logical_and(even, more))
        def _():
            _fire(c + 1, 1, sem_b)

        @pl.when(jnp.logical_and(jnp.logical_not(even), more))
        def _():
            _fire(c + 1, 0, sem_a)

        @pl.when(even)
        def _():
            _drain(0, sem_a)

        @pl.when(jnp.logical_not(even))
        def _():
            _drain(1, sem_b)

        srow = (c & 1) * CH * D
        uvec = uidx_v[pl.ds(c * CH, L)]
        mvec = midx_v[pl.ds(c * CH, L)]
        for k in range(CH):
            ulane = jnp.broadcast_to(uvec[k] & 127, (L,))
            mlane = jnp.broadcast_to(mvec[k] & 127, (L,))
            u_lo = plsc.load_gather(ublk_v, [srow + k * D + iota, ulane])
            u_hi = plsc.load_gather(ublk_v, [srow + k * D + HALF + iota, ulane])
            m_lo = plsc.load_gather(mblk_v, [srow + k * D + iota, mlane])
            m_hi = plsc.load_gather(mblk_v, [srow + k * D + HALF + iota, mlane])
            p = u_lo * m_lo + u_hi * m_hi
            plsc.store_scatter(prod_v, [iota * BPW + (c * CH + k)], p)
        return carry

    lax.fori_loop(0, NCH, chunk_body, 0)

    def group_body(g, carry):
        sl = pl.ds(g * L, L)
        acc = ub_v[sl] + mb_v[sl]
        for j in range(HALF):
            acc = acc + prod_v[pl.ds(j * BPW + g * L, L)]
        o_v[sl] = acc
        return carry

    lax.fori_loop(0, BPW // L, group_body, 0)

    pltpu.sync_copy(o_v, out_hbm.at[pl.ds(base, BPW)])


def kernel(user, mission, user_embedding, mission_embedding, user_bias, mission_bias):
    uembT = user_embedding.T
    membT = mission_embedding.T
    ub = user_bias.reshape(-1)
    mb = mission_bias.reshape(-1)
    return _mf_sc(user, mission, uembT, membT, ub, mb)
